# Initial kernel scaffold; baseline (speedup 1.0000x reference)
import jax
import jax.numpy as jnp
from jax.experimental import pallas as pl


def kernel(x, emb, W1, b1, W2, b2):
    raise NotImplementedError("write your pallas kernel here")


# speedup vs baseline: 2.1898x; 2.1898x over previous
import functools

import jax
import jax.numpy as jnp
from jax import lax
from jax.experimental import pallas as pl
from jax.experimental.pallas import tpu as pltpu
from jax.experimental.pallas import tpu_sc as plsc

B = 16384
S = 50
E = 32
HID = 32
NCLS = 10

NC = 2
NS = 16
NW = NC * NS
BPW = B // NW
CB = 2
IDXROW = CB * S
CHUNKS = BPW // CB

_mesh = plsc.VectorSubcoreMesh(
    core_axis_name="c", subcore_axis_name="s", num_cores=NC, num_subcores=NS
)


@functools.partial(
    pl.kernel,
    out_type=jax.ShapeDtypeStruct((B, E), jnp.float32),
    mesh=_mesh,
    scratch_types=[
        pltpu.VMEM((CHUNKS, IDXROW), jnp.int32),
        pltpu.VMEM((IDXROW, E), jnp.float32),
        pltpu.VMEM((BPW, E), jnp.float32),
        pltpu.SemaphoreType.DMA,
    ],
    compiler_params=pltpu.CompilerParams(use_tc_tiling_on_sc=False),
)
def _pool(x_hbm, emb_hbm, out_hbm, idx_v, rows_v, out_v, sem):
    wid = lax.axis_index("s") * NC + lax.axis_index("c")
    pltpu.sync_copy(x_hbm.at[wid], idx_v)

    def chunk_body(c, _):
        pltpu.async_copy(emb_hbm.at[idx_v.at[c]], rows_v, sem).wait()

        def tok_body(j, accs):
            a00, a01, a10, a11 = accs
            a00 = a00 + rows_v[j, pl.ds(0, 16)]
            a01 = a01 + rows_v[j, pl.ds(16, 16)]
            a10 = a10 + rows_v[S + j, pl.ds(0, 16)]
            a11 = a11 + rows_v[S + j, pl.ds(16, 16)]
            return (a00, a01, a10, a11)

        z = jnp.zeros((16,), jnp.float32)
        a00, a01, a10, a11 = lax.fori_loop(0, S, tok_body, (z, z, z, z))
        r0 = CB * c
        out_v[r0, pl.ds(0, 16)] = a00
        out_v[r0, pl.ds(16, 16)] = a01
        out_v[r0 + 1, pl.ds(0, 16)] = a10
        out_v[r0 + 1, pl.ds(16, 16)] = a11
        return 0

    lax.fori_loop(0, CHUNKS, chunk_body, 0)
    pltpu.sync_copy(out_v, out_hbm.at[pl.ds(wid * BPW, BPW)])


def _mlp_body(p_ref, w1_ref, b1_ref, w2_ref, b2_ref, o_ref):
    h = p_ref[...] * (1.0 / S)
    h = lax.dot_general(h, w1_ref[...], (((1,), (1,)), ((), ())),
                        preferred_element_type=jnp.float32)
    h = jnp.maximum(h + b1_ref[...], 0.0)
    o = lax.dot_general(h, w2_ref[...], (((1,), (1,)), ((), ())),
                        preferred_element_type=jnp.float32)
    o_ref[...] = o + b2_ref[...]


_BLK = 2048


def _mlp(pooled, W1, b1, W2, b2):
    grid = B // _BLK
    return pl.pallas_call(
        _mlp_body,
        out_shape=jax.ShapeDtypeStruct((B, NCLS), jnp.float32),
        grid=(grid,),
        in_specs=[
            pl.BlockSpec((_BLK, E), lambda i: (i, 0)),
            pl.BlockSpec((HID, E), lambda i: (0, 0)),
            pl.BlockSpec((1, HID), lambda i: (0, 0)),
            pl.BlockSpec((NCLS, HID), lambda i: (0, 0)),
            pl.BlockSpec((1, NCLS), lambda i: (0, 0)),
        ],
        out_specs=pl.BlockSpec((_BLK, NCLS), lambda i: (i, 0)),
    )(pooled, W1, b1, W2, b2)


def kernel(x, emb, W1, b1, W2, b2):
    xr = x.reshape(NW, CHUNKS, IDXROW)
    pooled = _pool(xr, emb)
    return _mlp(pooled, W1, b1.reshape(1, HID), W2, b2.reshape(1, NCLS))


# speedup vs baseline: 2.6520x; 1.2111x over previous
import functools

import jax
import jax.numpy as jnp
from jax import lax
from jax.experimental import pallas as pl
from jax.experimental.pallas import tpu as pltpu
from jax.experimental.pallas import tpu_sc as plsc

B = 16384
S = 50
E = 32
HID = 32
NCLS = 10

NC = 2
NS = 16
NW = NC * NS
BPW = B // NW
NBUF = 4

_mesh = plsc.VectorSubcoreMesh(
    core_axis_name="c", subcore_axis_name="s", num_cores=NC, num_subcores=NS
)


@functools.partial(
    pl.kernel,
    out_type=jax.ShapeDtypeStruct((B, E), jnp.float32),
    mesh=_mesh,
    scratch_types=[
        pltpu.VMEM((BPW, S), jnp.int32),
        pltpu.VMEM((NBUF, S, E), jnp.float32),
        pltpu.VMEM((BPW, E), jnp.float32),
        pltpu.SemaphoreType.DMA,
        pltpu.SemaphoreType.DMA,
        pltpu.SemaphoreType.DMA,
        pltpu.SemaphoreType.DMA,
    ],
    compiler_params=pltpu.CompilerParams(use_tc_tiling_on_sc=False),
)
def _pool(x_hbm, emb_hbm, out_hbm, idx_v, rows_v, out_v, s0, s1, s2, s3):
    sems = (s0, s1, s2, s3)
    wid = lax.axis_index("s") * NC + lax.axis_index("c")
    pltpu.sync_copy(x_hbm.at[pl.ds(wid * BPW, BPW)], idx_v)

    for b in range(NBUF):
        pltpu.async_copy(emb_hbm.at[idx_v.at[b]], rows_v.at[b], sems[b])

    def outer(g, _):
        r0 = g * NBUF
        for b in range(NBUF):
            r = r0 + b
            pltpu.make_async_copy(
                emb_hbm.at[pl.ds(0, S)], rows_v.at[b], sems[b]
            ).wait()
            rb = rows_v.at[b]
            a0 = rb[0, pl.ds(0, 16)]
            a1 = rb[0, pl.ds(16, 16)]
            c0 = rb[1, pl.ds(0, 16)]
            c1 = rb[1, pl.ds(16, 16)]
            for j in range(2, S, 2):
                a0 = a0 + rb[j, pl.ds(0, 16)]
                a1 = a1 + rb[j, pl.ds(16, 16)]
                c0 = c0 + rb[j + 1, pl.ds(0, 16)]
                c1 = c1 + rb[j + 1, pl.ds(16, 16)]
            out_v[r, pl.ds(0, 16)] = a0 + c0
            out_v[r, pl.ds(16, 16)] = a1 + c1

            nxt = r + NBUF

            @pl.when(nxt < BPW)
            def _():
                pltpu.async_copy(
                    emb_hbm.at[idx_v.at[nxt]], rows_v.at[b], sems[b]
                )

        return 0

    lax.fori_loop(0, BPW // NBUF, outer, 0)
    pltpu.sync_copy(out_v, out_hbm.at[pl.ds(wid * BPW, BPW)])


def _mlp_body(p_ref, w1_ref, b1_ref, w2_ref, b2_ref, o_ref):
    h = p_ref[...] * (1.0 / S)
    h = lax.dot_general(h, w1_ref[...], (((1,), (1,)), ((), ())),
                        preferred_element_type=jnp.float32)
    h = jnp.maximum(h + b1_ref[...], 0.0)
    o = lax.dot_general(h, w2_ref[...], (((1,), (1,)), ((), ())),
                        preferred_element_type=jnp.float32)
    o_ref[...] = o + b2_ref[...]


_BLK = 2048


def _mlp(pooled, W1, b1, W2, b2):
    grid = B // _BLK
    return pl.pallas_call(
        _mlp_body,
        out_shape=jax.ShapeDtypeStruct((B, NCLS), jnp.float32),
        grid=(grid,),
        in_specs=[
            pl.BlockSpec((_BLK, E), lambda i: (i, 0)),
            pl.BlockSpec((HID, E), lambda i: (0, 0)),
            pl.BlockSpec((1, HID), lambda i: (0, 0)),
            pl.BlockSpec((NCLS, HID), lambda i: (0, 0)),
            pl.BlockSpec((1, NCLS), lambda i: (0, 0)),
        ],
        out_specs=pl.BlockSpec((_BLK, NCLS), lambda i: (i, 0)),
    )(pooled, W1, b1, W2, b2)


def kernel(x, emb, W1, b1, W2, b2):
    pooled = _pool(x, emb)
    return _mlp(pooled, W1, b1.reshape(1, HID), W2, b2.reshape(1, NCLS))
